# baseline (device time: 207136 ns/iter reference)
import jax
import jax.numpy as jnp
from jax import lax
from jax.experimental import pallas as pl
from jax.experimental.pallas import tpu as pltpu

N_DEV = 16


def kernel(x, w_mat):
    m, k_per = x.shape
    k_per2, n = w_mat.shape
    assert k_per == k_per2
    m_blk = m // N_DEV

    def body(x_ref, w_ref, out_ref, send_ref, comm_ref, send_sem, recv_sems):
        my = lax.axis_index("i")
        left = (my - 1) % N_DEV
        right = (my + 1) % N_DEV

        barrier = pltpu.get_barrier_semaphore()
        for nbr in (left, right):
            pl.semaphore_signal(
                barrier, inc=1,
                device_id=(nbr,), device_id_type=pl.DeviceIdType.MESH,
            )
        pl.semaphore_wait(barrier, 2)

        def partial_chunk(c):
            rows = x_ref[pl.ds(c * m_blk, m_blk), :].astype(jnp.bfloat16)
            wb = w_ref[:, :].astype(jnp.bfloat16)
            return jnp.dot(rows, wb, preferred_element_type=jnp.float32)

        for s in range(N_DEV - 1):
            c = (my - s - 1) % N_DEV
            val = partial_chunk(c)
            if s > 0:
                val = val + comm_ref[s - 1]
            send_ref[:, :] = val
            rdma = pltpu.make_async_remote_copy(
                src_ref=send_ref,
                dst_ref=comm_ref.at[s],
                send_sem=send_sem,
                recv_sem=recv_sems.at[s],
                device_id=(right,),
                device_id_type=pl.DeviceIdType.MESH,
            )
            rdma.start()
            rdma.wait()

        out = comm_ref[N_DEV - 2] + partial_chunk(my)
        out_ref[:, :] = jnp.maximum(out, 0.0)

    return pl.pallas_call(
        body,
        out_shape=jax.ShapeDtypeStruct((m_blk, n), jnp.float32),
        in_specs=[
            pl.BlockSpec(memory_space=pltpu.VMEM),
            pl.BlockSpec(memory_space=pltpu.VMEM),
        ],
        out_specs=pl.BlockSpec(memory_space=pltpu.VMEM),
        scratch_shapes=[
            pltpu.VMEM((m_blk, n), jnp.float32),
            pltpu.VMEM((N_DEV - 1, m_blk, n), jnp.float32),
            pltpu.SemaphoreType.DMA,
            pltpu.SemaphoreType.DMA((N_DEV - 1,)),
        ],
        compiler_params=pltpu.CompilerParams(collective_id=0),
    )(x, w_mat)


# device time: 65566 ns/iter; 3.1592x vs baseline; 3.1592x over previous
import jax
import jax.numpy as jnp
from jax import lax
from jax.experimental import pallas as pl
from jax.experimental.pallas import tpu as pltpu

N_DEV = 16
N_PIECE = 2


def kernel(x, w_mat):
    m, k_per = x.shape
    k_per2, n = w_mat.shape
    assert k_per == k_per2
    m_blk = m // N_DEV
    nh = n // 2
    pw = nh // N_PIECE
    n_hop = N_DEV - 1

    def body(x_ref, w_ref, out_ref,
             send_r, send_l, comm_r, comm_l,
             ssem_r, ssem_l, rsem_r, rsem_l):
        my = lax.axis_index("i")
        left = (my - 1) % N_DEV
        right = (my + 1) % N_DEV

        barrier = pltpu.get_barrier_semaphore()
        for nbr in (left, right):
            pl.semaphore_signal(
                barrier, inc=1,
                device_id=(nbr,), device_id_type=pl.DeviceIdType.MESH,
            )
        pl.semaphore_wait(barrier, 2)

        def partial(c, col0):
            rows = x_ref[pl.ds(c * m_blk, m_blk), :].astype(jnp.bfloat16)
            wb = w_ref[:, col0:col0 + nh].astype(jnp.bfloat16)
            return jnp.dot(rows, wb, preferred_element_type=jnp.float32)

        dirs = {
            "r": (send_r, comm_r, ssem_r, rsem_r, right),
            "l": (send_l, comm_l, ssem_l, rsem_l, left),
        }
        prev = {}

        for s in range(n_hop):
            c_r = (my - s - 1) % N_DEV
            c_l = (my + s + 1) % N_DEV
            loc = {"r": partial(c_r, 0), "l": partial(c_l, nh)}

            for p in range(N_PIECE):
                for d in ("r", "l"):
                    sbuf, comm, ssem, rsem, tgt = dirs[d]
                    piece = loc[d][:, p * pw:(p + 1) * pw]
                    if s > 0:
                        prev[(d, p)].wait()
                        piece = piece + comm[s - 1, p].astype(jnp.float32)
                    sbuf[p] = piece.astype(jnp.bfloat16)
                    rdma = pltpu.make_async_remote_copy(
                        src_ref=sbuf.at[p],
                        dst_ref=comm.at[s, p],
                        send_sem=ssem.at[p],
                        recv_sem=rsem.at[s, p],
                        device_id=(tgt,),
                        device_id_type=pl.DeviceIdType.MESH,
                    )
                    rdma.start()
                    prev[(d, p)] = rdma

        own = {"r": partial(my, 0), "l": partial(my, nh)}
        for p in range(N_PIECE):
            for d in ("r", "l"):
                _, comm, _, _, _ = dirs[d]
                prev[(d, p)].wait()
                col0 = (0 if d == "r" else nh) + p * pw
                acc = own[d][:, p * pw:(p + 1) * pw] \
                    + comm[n_hop - 1, p].astype(jnp.float32)
                out_ref[:, col0:col0 + pw] = jnp.maximum(acc, 0.0)

    return pl.pallas_call(
        body,
        out_shape=jax.ShapeDtypeStruct((m_blk, n), jnp.float32),
        in_specs=[
            pl.BlockSpec(memory_space=pltpu.VMEM),
            pl.BlockSpec(memory_space=pltpu.VMEM),
        ],
        out_specs=pl.BlockSpec(memory_space=pltpu.VMEM),
        scratch_shapes=[
            pltpu.VMEM((N_PIECE, m_blk, pw), jnp.bfloat16),
            pltpu.VMEM((N_PIECE, m_blk, pw), jnp.bfloat16),
            pltpu.VMEM((n_hop, N_PIECE, m_blk, pw), jnp.bfloat16),
            pltpu.VMEM((n_hop, N_PIECE, m_blk, pw), jnp.bfloat16),
            pltpu.SemaphoreType.DMA((N_PIECE,)),
            pltpu.SemaphoreType.DMA((N_PIECE,)),
            pltpu.SemaphoreType.DMA((n_hop, N_PIECE)),
            pltpu.SemaphoreType.DMA((n_hop, N_PIECE)),
        ],
        compiler_params=pltpu.CompilerParams(collective_id=0),
    )(x, w_mat)


# device time: 63934 ns/iter; 3.2398x vs baseline; 1.0255x over previous
import jax
import jax.numpy as jnp
from jax import lax
from jax.experimental import pallas as pl
from jax.experimental.pallas import tpu as pltpu

N_DEV = 16
N_PIECE = 4


def kernel(x, w_mat):
    m, k_per = x.shape
    k_per2, n = w_mat.shape
    assert k_per == k_per2
    m_blk = m // N_DEV
    nh = n // 2
    pw = nh // N_PIECE
    n_hop = N_DEV - 1

    def body(x_ref, w_ref, out_ref,
             send_r, send_l, comm_r, comm_l,
             ssem_r, ssem_l, rsem_r, rsem_l):
        my = lax.axis_index("i")
        left = (my - 1) % N_DEV
        right = (my + 1) % N_DEV

        barrier = pltpu.get_barrier_semaphore()
        for nbr in (left, right):
            pl.semaphore_signal(
                barrier, inc=1,
                device_id=(nbr,), device_id_type=pl.DeviceIdType.MESH,
            )
        pl.semaphore_wait(barrier, 2)

        def partial(c, col0):
            rows = x_ref[pl.ds(c * m_blk, m_blk), :].astype(jnp.bfloat16)
            wb = w_ref[:, col0:col0 + nh].astype(jnp.bfloat16)
            return jnp.dot(rows, wb, preferred_element_type=jnp.float32)

        dirs = {
            "r": (send_r, comm_r, ssem_r, rsem_r, right),
            "l": (send_l, comm_l, ssem_l, rsem_l, left),
        }
        prev = {}

        for s in range(n_hop):
            c_r = (my - s - 1) % N_DEV
            c_l = (my + s + 1) % N_DEV
            loc = {"r": partial(c_r, 0), "l": partial(c_l, nh)}

            for p in range(N_PIECE):
                for d in ("r", "l"):
                    sbuf, comm, ssem, rsem, tgt = dirs[d]
                    piece = loc[d][:, p * pw:(p + 1) * pw]
                    if s > 0:
                        prev[(d, p)].wait()
                        piece = piece + comm[s - 1, p].astype(jnp.float32)
                    sbuf[p] = piece.astype(jnp.bfloat16)
                    rdma = pltpu.make_async_remote_copy(
                        src_ref=sbuf.at[p],
                        dst_ref=comm.at[s, p],
                        send_sem=ssem.at[p],
                        recv_sem=rsem.at[s, p],
                        device_id=(tgt,),
                        device_id_type=pl.DeviceIdType.MESH,
                    )
                    rdma.start()
                    prev[(d, p)] = rdma

        own = {"r": partial(my, 0), "l": partial(my, nh)}
        for p in range(N_PIECE):
            for d in ("r", "l"):
                _, comm, _, _, _ = dirs[d]
                prev[(d, p)].wait()
                col0 = (0 if d == "r" else nh) + p * pw
                acc = own[d][:, p * pw:(p + 1) * pw] \
                    + comm[n_hop - 1, p].astype(jnp.float32)
                out_ref[:, col0:col0 + pw] = jnp.maximum(acc, 0.0)

    return pl.pallas_call(
        body,
        out_shape=jax.ShapeDtypeStruct((m_blk, n), jnp.float32),
        in_specs=[
            pl.BlockSpec(memory_space=pltpu.VMEM),
            pl.BlockSpec(memory_space=pltpu.VMEM),
        ],
        out_specs=pl.BlockSpec(memory_space=pltpu.VMEM),
        scratch_shapes=[
            pltpu.VMEM((N_PIECE, m_blk, pw), jnp.bfloat16),
            pltpu.VMEM((N_PIECE, m_blk, pw), jnp.bfloat16),
            pltpu.VMEM((n_hop, N_PIECE, m_blk, pw), jnp.bfloat16),
            pltpu.VMEM((n_hop, N_PIECE, m_blk, pw), jnp.bfloat16),
            pltpu.SemaphoreType.DMA((N_PIECE,)),
            pltpu.SemaphoreType.DMA((N_PIECE,)),
            pltpu.SemaphoreType.DMA((n_hop, N_PIECE)),
            pltpu.SemaphoreType.DMA((n_hop, N_PIECE)),
        ],
        compiler_params=pltpu.CompilerParams(collective_id=0),
    )(x, w_mat)


# device time: 63871 ns/iter; 3.2430x vs baseline; 1.0010x over previous
import jax
import jax.numpy as jnp
from jax import lax
from jax.experimental import pallas as pl
from jax.experimental.pallas import tpu as pltpu

N_DEV = 16
N_PIECE = 4


def kernel(x, w_mat):
    m, k_per = x.shape
    k_per2, n = w_mat.shape
    assert k_per == k_per2
    m_blk = m // N_DEV
    nh = n // 2
    pw = nh // N_PIECE
    n_hop = N_DEV - 1

    def body(x_ref, w_ref, out_ref,
             send_r, send_l, comm_r, comm_l,
             ssem_r, ssem_l, rsem_r, rsem_l):
        my = lax.axis_index("i")
        left = (my - 1) % N_DEV
        right = (my + 1) % N_DEV

        barrier = pltpu.get_barrier_semaphore()
        for nbr in (left, right):
            pl.semaphore_signal(
                barrier, inc=1,
                device_id=(nbr,), device_id_type=pl.DeviceIdType.MESH,
            )
        pl.semaphore_wait(barrier, 2)

        def partial(c, col0):
            rows = x_ref[pl.ds(c * m_blk, m_blk), :].astype(jnp.bfloat16)
            wb = w_ref[:, col0:col0 + nh].astype(jnp.bfloat16)
            return jnp.dot(rows, wb, preferred_element_type=jnp.float32)

        dirs = {
            "r": (send_r, comm_r, ssem_r, rsem_r, right),
            "l": (send_l, comm_l, ssem_l, rsem_l, left),
        }
        prev = {}

        for s in range(n_hop):
            c_r = (my - s - 1) % N_DEV
            c_l = (my + s + 1) % N_DEV
            loc = {
                "r": partial(c_r, 0).astype(jnp.bfloat16),
                "l": partial(c_l, nh).astype(jnp.bfloat16),
            }

            for p in range(N_PIECE):
                for d in ("r", "l"):
                    sbuf, comm, ssem, rsem, tgt = dirs[d]
                    piece = loc[d][:, p * pw:(p + 1) * pw]
                    if s > 0:
                        prev[(d, p)].wait()
                        piece = piece + comm[s - 1, p]
                    sbuf[p] = piece
                    rdma = pltpu.make_async_remote_copy(
                        src_ref=sbuf.at[p],
                        dst_ref=comm.at[s, p],
                        send_sem=ssem.at[p],
                        recv_sem=rsem.at[s, p],
                        device_id=(tgt,),
                        device_id_type=pl.DeviceIdType.MESH,
                    )
                    rdma.start()
                    prev[(d, p)] = rdma

        own = {"r": partial(my, 0), "l": partial(my, nh)}
        for p in range(N_PIECE):
            for d in ("r", "l"):
                _, comm, _, _, _ = dirs[d]
                prev[(d, p)].wait()
                col0 = (0 if d == "r" else nh) + p * pw
                acc = own[d][:, p * pw:(p + 1) * pw] \
                    + comm[n_hop - 1, p].astype(jnp.float32)
                out_ref[:, col0:col0 + pw] = jnp.maximum(acc, 0.0)

    return pl.pallas_call(
        body,
        out_shape=jax.ShapeDtypeStruct((m_blk, n), jnp.float32),
        in_specs=[
            pl.BlockSpec(memory_space=pltpu.VMEM),
            pl.BlockSpec(memory_space=pltpu.VMEM),
        ],
        out_specs=pl.BlockSpec(memory_space=pltpu.VMEM),
        scratch_shapes=[
            pltpu.VMEM((N_PIECE, m_blk, pw), jnp.bfloat16),
            pltpu.VMEM((N_PIECE, m_blk, pw), jnp.bfloat16),
            pltpu.VMEM((n_hop, N_PIECE, m_blk, pw), jnp.bfloat16),
            pltpu.VMEM((n_hop, N_PIECE, m_blk, pw), jnp.bfloat16),
            pltpu.SemaphoreType.DMA((N_PIECE,)),
            pltpu.SemaphoreType.DMA((N_PIECE,)),
            pltpu.SemaphoreType.DMA((n_hop, N_PIECE)),
            pltpu.SemaphoreType.DMA((n_hop, N_PIECE)),
        ],
        compiler_params=pltpu.CompilerParams(collective_id=0),
    )(x, w_mat)


# device time: 56419 ns/iter; 3.6714x vs baseline; 1.1321x over previous
import jax
import jax.numpy as jnp
from jax import lax
from jax.experimental import pallas as pl
from jax.experimental.pallas import tpu as pltpu

N_DEV = 16
N_PIECE = 2

PERM = [0, 4, 8, 12, 13, 9, 5, 1, 2, 6, 10, 14, 15, 11, 7, 3]
INV_PERM = [PERM.index(i) for i in range(N_DEV)]


def kernel(x, w_mat):
    m, k_per = x.shape
    k_per2, n = w_mat.shape
    assert k_per == k_per2
    m_blk = m // N_DEV
    nh = n // 2
    pw = nh // N_PIECE
    n_hop = N_DEV - 1

    def body(x_ref, w_ref, out_ref,
             send_r, send_l, comm_r, comm_l,
             ssem_r, ssem_l, rsem_r, rsem_l):
        my = lax.axis_index("i")

        def lut(idx, table):
            v = jnp.int32(table[0])
            for i in range(1, N_DEV):
                v = jnp.where(idx == i, jnp.int32(table[i]), v)
            return v

        r_pos = lut(my, INV_PERM)
        right = lut((r_pos + 1) % N_DEV, PERM)
        left = lut((r_pos - 1) % N_DEV, PERM)

        barrier = pltpu.get_barrier_semaphore()
        for nbr in (left, right):
            pl.semaphore_signal(
                barrier, inc=1,
                device_id=(nbr,), device_id_type=pl.DeviceIdType.MESH,
            )
        pl.semaphore_wait(barrier, 2)

        def partial(c, col0):
            rows = x_ref[pl.ds(c * m_blk, m_blk), :].astype(jnp.bfloat16)
            wb = w_ref[:, col0:col0 + nh].astype(jnp.bfloat16)
            return jnp.dot(rows, wb, preferred_element_type=jnp.float32)

        dirs = {
            "r": (send_r, comm_r, ssem_r, rsem_r, right),
            "l": (send_l, comm_l, ssem_l, rsem_l, left),
        }
        prev = {}

        for s in range(n_hop):
            c_r = lut((r_pos - s - 1) % N_DEV, PERM)
            c_l = lut((r_pos + s + 1) % N_DEV, PERM)
            loc = {
                "r": partial(c_r, 0).astype(jnp.bfloat16),
                "l": partial(c_l, nh).astype(jnp.bfloat16),
            }

            for p in range(N_PIECE):
                for d in ("r", "l"):
                    sbuf, comm, ssem, rsem, tgt = dirs[d]
                    piece = loc[d][:, p * pw:(p + 1) * pw]
                    if s > 0:
                        prev[(d, p)].wait()
                        piece = piece + comm[s - 1, p]
                    sbuf[p] = piece
                    rdma = pltpu.make_async_remote_copy(
                        src_ref=sbuf.at[p],
                        dst_ref=comm.at[s, p],
                        send_sem=ssem.at[p],
                        recv_sem=rsem.at[s, p],
                        device_id=(tgt,),
                        device_id_type=pl.DeviceIdType.MESH,
                    )
                    rdma.start()
                    prev[(d, p)] = rdma

        own = {"r": partial(my, 0), "l": partial(my, nh)}
        for p in range(N_PIECE):
            for d in ("r", "l"):
                _, comm, _, _, _ = dirs[d]
                prev[(d, p)].wait()
                col0 = (0 if d == "r" else nh) + p * pw
                acc = own[d][:, p * pw:(p + 1) * pw] \
                    + comm[n_hop - 1, p].astype(jnp.float32)
                out_ref[:, col0:col0 + pw] = jnp.maximum(acc, 0.0)

    return pl.pallas_call(
        body,
        out_shape=jax.ShapeDtypeStruct((m_blk, n), jnp.float32),
        in_specs=[
            pl.BlockSpec(memory_space=pltpu.VMEM),
            pl.BlockSpec(memory_space=pltpu.VMEM),
        ],
        out_specs=pl.BlockSpec(memory_space=pltpu.VMEM),
        scratch_shapes=[
            pltpu.VMEM((N_PIECE, m_blk, pw), jnp.bfloat16),
            pltpu.VMEM((N_PIECE, m_blk, pw), jnp.bfloat16),
            pltpu.VMEM((n_hop, N_PIECE, m_blk, pw), jnp.bfloat16),
            pltpu.VMEM((n_hop, N_PIECE, m_blk, pw), jnp.bfloat16),
            pltpu.SemaphoreType.DMA((N_PIECE,)),
            pltpu.SemaphoreType.DMA((N_PIECE,)),
            pltpu.SemaphoreType.DMA((n_hop, N_PIECE)),
            pltpu.SemaphoreType.DMA((n_hop, N_PIECE)),
        ],
        compiler_params=pltpu.CompilerParams(collective_id=0),
    )(x, w_mat)


# device time: 51981 ns/iter; 3.9848x vs baseline; 1.0854x over previous
import jax
import jax.numpy as jnp
from jax import lax
from jax.experimental import pallas as pl
from jax.experimental.pallas import tpu as pltpu

N_DEV = 16
N_PIECE = 4

PERM = [0, 4, 8, 12, 13, 9, 5, 1, 2, 6, 10, 14, 15, 11, 7, 3]
INV_PERM = [PERM.index(i) for i in range(N_DEV)]


def kernel(x, w_mat):
    m, k_per = x.shape
    k_per2, n = w_mat.shape
    assert k_per == k_per2
    m_blk = m // N_DEV
    nh = n // 2
    pw = nh // N_PIECE
    n_hop = N_DEV - 1

    def body(x_ref, w_ref, out_ref,
             send_r, send_l, comm_r, comm_l,
             ssem_r, ssem_l, rsem_r, rsem_l):
        my = lax.axis_index("i")

        def lut(idx, table):
            v = jnp.int32(table[0])
            for i in range(1, N_DEV):
                v = jnp.where(idx == i, jnp.int32(table[i]), v)
            return v

        r_pos = lut(my, INV_PERM)
        right = lut((r_pos + 1) % N_DEV, PERM)
        left = lut((r_pos - 1) % N_DEV, PERM)

        barrier = pltpu.get_barrier_semaphore()
        for nbr in (left, right):
            pl.semaphore_signal(
                barrier, inc=1,
                device_id=(nbr,), device_id_type=pl.DeviceIdType.MESH,
            )
        pl.semaphore_wait(barrier, 2)

        def partial(c, col0):
            rows = x_ref[pl.ds(c * m_blk, m_blk), :].astype(jnp.bfloat16)
            wb = w_ref[:, col0:col0 + nh].astype(jnp.bfloat16)
            return jnp.dot(rows, wb, preferred_element_type=jnp.float32)

        dirs = {
            "r": (send_r, comm_r, ssem_r, rsem_r, right),
            "l": (send_l, comm_l, ssem_l, rsem_l, left),
        }
        prev = {}

        for s in range(n_hop):
            c_r = lut((r_pos - s - 1) % N_DEV, PERM)
            c_l = lut((r_pos + s + 1) % N_DEV, PERM)
            loc = {
                "r": partial(c_r, 0).astype(jnp.bfloat16),
                "l": partial(c_l, nh).astype(jnp.bfloat16),
            }

            for p in range(N_PIECE):
                for d in ("r", "l"):
                    sbuf, comm, ssem, rsem, tgt = dirs[d]
                    piece = loc[d][:, p * pw:(p + 1) * pw]
                    if s > 0:
                        prev[(d, p)].wait()
                        piece = piece + comm[s - 1, p]
                    sbuf[p] = piece
                    rdma = pltpu.make_async_remote_copy(
                        src_ref=sbuf.at[p],
                        dst_ref=comm.at[s, p],
                        send_sem=ssem.at[p],
                        recv_sem=rsem.at[s, p],
                        device_id=(tgt,),
                        device_id_type=pl.DeviceIdType.MESH,
                    )
                    rdma.start()
                    prev[(d, p)] = rdma

        own = {"r": partial(my, 0), "l": partial(my, nh)}
        for p in range(N_PIECE):
            for d in ("r", "l"):
                _, comm, _, _, _ = dirs[d]
                prev[(d, p)].wait()
                col0 = (0 if d == "r" else nh) + p * pw
                acc = own[d][:, p * pw:(p + 1) * pw] \
                    + comm[n_hop - 1, p].astype(jnp.float32)
                out_ref[:, col0:col0 + pw] = jnp.maximum(acc, 0.0)

    return pl.pallas_call(
        body,
        out_shape=jax.ShapeDtypeStruct((m_blk, n), jnp.float32),
        in_specs=[
            pl.BlockSpec(memory_space=pltpu.VMEM),
            pl.BlockSpec(memory_space=pltpu.VMEM),
        ],
        out_specs=pl.BlockSpec(memory_space=pltpu.VMEM),
        scratch_shapes=[
            pltpu.VMEM((N_PIECE, m_blk, pw), jnp.bfloat16),
            pltpu.VMEM((N_PIECE, m_blk, pw), jnp.bfloat16),
            pltpu.VMEM((n_hop, N_PIECE, m_blk, pw), jnp.bfloat16),
            pltpu.VMEM((n_hop, N_PIECE, m_blk, pw), jnp.bfloat16),
            pltpu.SemaphoreType.DMA((N_PIECE,)),
            pltpu.SemaphoreType.DMA((N_PIECE,)),
            pltpu.SemaphoreType.DMA((n_hop, N_PIECE)),
            pltpu.SemaphoreType.DMA((n_hop, N_PIECE)),
        ],
        compiler_params=pltpu.CompilerParams(collective_id=0),
    )(x, w_mat)
